# Initial kernel scaffold; baseline (speedup 1.0000x reference)
import jax
import jax.numpy as jnp
from jax.experimental import pallas as pl


def kernel(x, idx, x1):
    raise NotImplementedError("write your pallas kernel here")


# speedup vs baseline: 76.8386x; 76.8386x over previous
import functools

import jax
import jax.numpy as jnp
from jax import lax
from jax.experimental import pallas as pl
from jax.experimental.pallas import tpu as pltpu
from jax.experimental.pallas import tpu_sc as plsc

L = 16


def _make_unpool(P, N, M):
    info = plsc.get_sparse_core_info()
    nc, ns = info.num_cores, info.num_subcores
    nw = nc * ns
    assert P % nw == 0
    pp = P // nw

    mesh = plsc.VectorSubcoreMesh(core_axis_name="c", subcore_axis_name="s")

    @functools.partial(
        pl.kernel,
        mesh=mesh,
        compiler_params=pltpu.CompilerParams(needs_layout_passes=False),
        out_type=jax.ShapeDtypeStruct((P, M), jnp.float32),
        scratch_types=[
            pltpu.VMEM((N,), jnp.int32),
            pltpu.VMEM((N,), jnp.float32),
            pltpu.VMEM((M,), jnp.float32),
        ],
    )
    def k(x_hbm, idx_hbm, out_hbm, ibuf, xbuf, obuf):
        wid = lax.axis_index("s") * nc + lax.axis_index("c")
        zeros = jnp.zeros((L,), jnp.float32)

        def zbody(i, c):
            obuf[pl.ds(i * L, L)] = zeros
            return c

        lax.fori_loop(0, M // L, zbody, 0)

        def plane_body(j, c):
            p = wid * pp + j
            pltpu.sync_copy(idx_hbm.at[p], ibuf)
            pltpu.sync_copy(x_hbm.at[p], xbuf)

            def sbody(i, c):
                iv = ibuf[pl.ds(i * L, L)]
                xv = xbuf[pl.ds(i * L, L)]
                plsc.store_scatter(obuf, [iv], xv)
                return c

            lax.fori_loop(0, N // L, sbody, 0)
            pltpu.sync_copy(obuf, out_hbm.at[p])

            def zsbody(i, c):
                iv = ibuf[pl.ds(i * L, L)]
                plsc.store_scatter(obuf, [iv], zeros)
                return c

            lax.fori_loop(0, N // L, zsbody, 0)
            return c

        lax.fori_loop(0, pp, plane_body, 0)

    return k


def kernel(x, idx, x1):
    B, C, Hp, Wp = x.shape
    _, _, H, W = x1.shape
    P, N, M = B * C, Hp * Wp, H * W
    unpool = _make_unpool(P, N, M)
    out2 = unpool(x.reshape(P, N), idx.reshape(P, N))
    return out2.reshape(B, C, H, W)
